# baseline (speedup 1.0000x reference)
import jax
import jax.numpy as jnp
from jax.experimental import pallas as pl
from jax.experimental.pallas import tpu as pltpu

N = 1024
C = 1024
KV = 128
KT = 128
KSEL = KV + KT
K2 = int((N - KSEL) * 0.25)
NOUT = KSEL + K2
SCALE = C ** -0.5
T = 4
TC = C // T


def _body(ar_ref, ac_ref, sr_ref, sc_ref, key_ref, img_ref, out_ref,
          q_scr, cos_scr, nb2_scr, nk2_scr, cmask_scr, score_scr):
    f32 = jnp.float32
    i = pl.program_id(0)

    @pl.when(i == 0)
    def _selection():
        ar = ar_ref[...]
        ac = ac_ref[...]
        sr = sr_ref[...]
        sc = sc_ref[...]

        ioj = jax.lax.broadcasted_iota(jnp.int32, (N, N), 0)
        ioi = jax.lax.broadcasted_iota(jnp.int32, (N, N), 1)
        ident = (ioj == ioi).astype(f32)

        def rank_row(vc, vr):
            m = (vc > vr) | ((vc == vr) & (ioj < ioi))
            return jnp.sum(m.astype(f32), axis=0, keepdims=True)

        def to_col(vr):
            return jax.lax.dot_general(ident, vr, (((1,), (1,)), ((), ())),
                                       preferred_element_type=f32)

        rv_r = rank_row(ac, ar)
        rt_r = rank_row(sc, sr)
        sel_r = ((rv_r < KV) | (rt_r < KT)).astype(f32)
        k1_r = sel_r * 1e6 + ar
        k1_c = to_col(k1_r)
        rs_r = rank_row(k1_c, k1_r)
        a_r = rs_r < KSEL
        a_c = to_col(a_r.astype(f32)) > 0.5

        m2 = (~a_c) & ((ac > ar) | ((ac == ar) & (ioj < ioi)))
        r2_r = jnp.sum(m2.astype(f32), axis=0, keepdims=True)
        b_r = (~a_r) & (r2_r < K2)
        cmask = (~a_r) & (~b_r)

        row_of = jnp.where(a_r, rs_r, jnp.where(b_r, KSEL + r2_r, 2.0 * N))
        io_out = jax.lax.broadcasted_iota(jnp.int32, (NOUT, N), 0)
        q_scr[0:NOUT, :] = (io_out == row_of.astype(jnp.int32)).astype(f32)

        cmask_scr[...] = cmask.astype(f32)
        neg = jnp.float32(-jnp.inf)
        t = jnp.where(cmask, 50.0 * sr, neg)
        te = jnp.exp(t - jnp.max(t, axis=1, keepdims=True))
        sm = te / jnp.sum(te, axis=1, keepdims=True)
        score_scr[...] = ar * sm

        cos_scr[...] = jnp.zeros((K2, N), f32)
        nb2_scr[...] = jnp.zeros((K2, 1), f32)
        nk2_scr[...] = jnp.zeros((N, 1), f32)

    @pl.when((i >= 1) & (i <= T))
    def _key_sweep():
        key_t = key_ref[...]
        kb_t = jax.lax.dot_general(q_scr[KSEL:NOUT, :], key_t,
                                   (((1,), (0,)), ((), ())),
                                   preferred_element_type=f32)
        cos_scr[...] += jax.lax.dot_general(kb_t, key_t,
                                            (((1,), (1,)), ((), ())),
                                            preferred_element_type=f32)
        nb2_scr[...] += jnp.sum(kb_t * kb_t, axis=1, keepdims=True)
        nk2_scr[...] += jnp.sum(key_t * key_t, axis=1, keepdims=True)

    @pl.when(i == T + 1)
    def _finalize_weights():
        ioj = jax.lax.broadcasted_iota(jnp.int32, (N, N), 0)
        ioi = jax.lax.broadcasted_iota(jnp.int32, (N, N), 1)
        ident = (ioj == ioi).astype(f32)
        nk2_r = jax.lax.dot_general(nk2_scr[...], ident,
                                    (((0,), (0,)), ((), ())),
                                    preferred_element_type=f32)
        nb = jnp.maximum(jnp.sqrt(nb2_scr[...]), 1e-12)
        nk = jnp.maximum(jnp.sqrt(nk2_r), 1e-12)
        cos = cos_scr[...] / nb / nk * SCALE
        neg = jnp.float32(-jnp.inf)
        logits = jnp.where(cmask_scr[...] > 0.5, cos, neg)
        e = jnp.exp(logits - jnp.max(logits, axis=1, keepdims=True))
        w = e / jnp.sum(e, axis=1, keepdims=True)
        q_scr[NOUT:NOUT + K2, :] = w * score_scr[...]

    @pl.when(i >= T + 1)
    def _img_sweep():
        img_t = img_ref[...]
        y = jax.lax.dot_general(q_scr[...], img_t, (((1,), (0,)), ((), ())),
                                preferred_element_type=f32)
        out_ref[0:KSEL, :] = y[0:KSEL, :]
        out_ref[KSEL:NOUT, :] = y[KSEL:NOUT, :] + y[NOUT:NOUT + K2, :]


def kernel(image_features, key_features, cls_attn, similarity):
    img = image_features[0]
    key = key_features[0]
    ar = cls_attn
    ac = cls_attn.reshape(N, 1)
    sr = similarity
    sc = similarity.reshape(N, 1)
    f32 = jnp.float32
    out = pl.pallas_call(
        _body,
        grid=(2 * T + 1,),
        in_specs=[
            pl.BlockSpec((1, N), lambda i: (0, 0)),
            pl.BlockSpec((N, 1), lambda i: (0, 0)),
            pl.BlockSpec((1, N), lambda i: (0, 0)),
            pl.BlockSpec((N, 1), lambda i: (0, 0)),
            pl.BlockSpec((N, TC), lambda i: (0, jnp.clip(i - 1, 0, T - 1))),
            pl.BlockSpec((N, TC), lambda i: (0, jnp.clip(i - T - 1, 0, T - 1))),
        ],
        out_specs=pl.BlockSpec((NOUT, TC), lambda i: (0, jnp.clip(i - T - 1, 0, T - 1))),
        out_shape=jax.ShapeDtypeStruct((NOUT, C), f32),
        scratch_shapes=[
            pltpu.VMEM((NOUT + K2, N), f32),
            pltpu.VMEM((K2, N), f32),
            pltpu.VMEM((K2, 1), f32),
            pltpu.VMEM((N, 1), f32),
            pltpu.VMEM((1, N), f32),
            pltpu.VMEM((1, N), f32),
        ],
    )(ar, ac, sr, sc, key, img)
    return out[None]


# speedup vs baseline: 1.1566x; 1.1566x over previous
import jax
import jax.numpy as jnp
from jax.experimental import pallas as pl
from jax.experimental.pallas import tpu as pltpu

N = 1024
C = 1024
KV = 128
KT = 128
KSEL = KV + KT
K2 = int((N - KSEL) * 0.25)
NOUT = KSEL + K2
SCALE = C ** -0.5
T = 4
TC = C // T


def _body(ar_ref, ac_ref, sr_ref, sc_ref, key_hbm, img_hbm, out_hbm,
          key_v, img_v, out_v, q_scr, in_sem, out_sem):
    f32 = jnp.float32

    key_cp = [pltpu.make_async_copy(key_hbm.at[:, t * TC:(t + 1) * TC],
                                    key_v.at[:, t * TC:(t + 1) * TC],
                                    in_sem.at[t]) for t in range(T)]
    img_cp = [pltpu.make_async_copy(img_hbm.at[:, t * TC:(t + 1) * TC],
                                    img_v.at[:, t * TC:(t + 1) * TC],
                                    in_sem.at[T + t]) for t in range(T)]
    for cp in key_cp:
        cp.start()
    for cp in img_cp:
        cp.start()

    ar = ar_ref[...]
    ac = ac_ref[...]
    sr = sr_ref[...]
    sc = sc_ref[...]

    ioj = jax.lax.broadcasted_iota(jnp.int32, (N, N), 0)
    ioi = jax.lax.broadcasted_iota(jnp.int32, (N, N), 1)
    ident = (ioj == ioi).astype(f32)

    def rank_row(vc, vr):
        m = (vc > vr) | ((vc == vr) & (ioj < ioi))
        return jnp.sum(m.astype(f32), axis=0, keepdims=True)

    def to_col(vr):
        return jax.lax.dot_general(ident, vr, (((1,), (1,)), ((), ())),
                                   preferred_element_type=f32)

    rv_r = rank_row(ac, ar)
    rt_r = rank_row(sc, sr)
    sel_r = ((rv_r < KV) | (rt_r < KT)).astype(f32)
    k1_r = sel_r * 1e6 + ar
    k1_c = to_col(k1_r)
    rs_r = rank_row(k1_c, k1_r)
    a_r = rs_r < KSEL
    a_c = to_col(a_r.astype(f32)) > 0.5

    m2 = (~a_c) & ((ac > ar) | ((ac == ar) & (ioj < ioi)))
    r2_r = jnp.sum(m2.astype(f32), axis=0, keepdims=True)
    b_r = (~a_r) & (r2_r < K2)
    cmask = (~a_r) & (~b_r)

    row_of = jnp.where(a_r, rs_r, jnp.where(b_r, KSEL + r2_r, 2.0 * N))
    io_out = jax.lax.broadcasted_iota(jnp.int32, (NOUT, N), 0)
    q_scr[0:NOUT, :] = (io_out == row_of.astype(jnp.int32)).astype(f32)

    neg = jnp.float32(-jnp.inf)
    t_log = jnp.where(cmask, 50.0 * sr, neg)
    te = jnp.exp(t_log - jnp.max(t_log, axis=1, keepdims=True))
    sm = te / jnp.sum(te, axis=1, keepdims=True)
    score = ar * sm

    cos = jnp.zeros((K2, N), f32)
    nb2 = jnp.zeros((K2, 1), f32)
    nk2 = jnp.zeros((N, 1), f32)
    for t in range(T):
        key_cp[t].wait()
        key_t = key_v[:, t * TC:(t + 1) * TC]
        kb_t = jax.lax.dot_general(q_scr[KSEL:NOUT, :], key_t,
                                   (((1,), (0,)), ((), ())),
                                   preferred_element_type=f32)
        cos += jax.lax.dot_general(kb_t, key_t, (((1,), (1,)), ((), ())),
                                   preferred_element_type=f32)
        nb2 += jnp.sum(kb_t * kb_t, axis=1, keepdims=True)
        nk2 += jnp.sum(key_t * key_t, axis=1, keepdims=True)

    nk2_r = jax.lax.dot_general(nk2, ident, (((0,), (0,)), ((), ())),
                                preferred_element_type=f32)
    nb = jnp.maximum(jnp.sqrt(nb2), 1e-12)
    nk = jnp.maximum(jnp.sqrt(nk2_r), 1e-12)
    cosn = cos / nb / nk * SCALE
    logits = jnp.where(cmask, cosn, neg)
    e = jnp.exp(logits - jnp.max(logits, axis=1, keepdims=True))
    w = e / jnp.sum(e, axis=1, keepdims=True)
    q_scr[NOUT:NOUT + K2, :] = w * score

    out_cp = []
    for t in range(T):
        img_cp[t].wait()
        img_t = img_v[:, t * TC:(t + 1) * TC]
        y = jax.lax.dot_general(q_scr[...], img_t, (((1,), (0,)), ((), ())),
                                preferred_element_type=f32)
        out_v[0:KSEL, t * TC:(t + 1) * TC] = y[0:KSEL, :]
        out_v[KSEL:NOUT, t * TC:(t + 1) * TC] = (y[KSEL:NOUT, :] +
                                                 y[NOUT:NOUT + K2, :])
        cp = pltpu.make_async_copy(out_v.at[:, t * TC:(t + 1) * TC],
                                   out_hbm.at[:, t * TC:(t + 1) * TC],
                                   out_sem.at[t])
        cp.start()
        out_cp.append(cp)
    for cp in out_cp:
        cp.wait()


def kernel(image_features, key_features, cls_attn, similarity):
    img = image_features[0]
    key = key_features[0]
    ar = cls_attn
    ac = cls_attn.reshape(N, 1)
    sr = similarity
    sc = similarity.reshape(N, 1)
    f32 = jnp.float32
    out = pl.pallas_call(
        _body,
        in_specs=[
            pl.BlockSpec((1, N), lambda: (0, 0)),
            pl.BlockSpec((N, 1), lambda: (0, 0)),
            pl.BlockSpec((1, N), lambda: (0, 0)),
            pl.BlockSpec((N, 1), lambda: (0, 0)),
            pl.BlockSpec(memory_space=pl.ANY),
            pl.BlockSpec(memory_space=pl.ANY),
        ],
        out_specs=pl.BlockSpec(memory_space=pl.ANY),
        out_shape=jax.ShapeDtypeStruct((NOUT, C), f32),
        scratch_shapes=[
            pltpu.VMEM((N, C), f32),
            pltpu.VMEM((N, C), f32),
            pltpu.VMEM((NOUT, C), f32),
            pltpu.VMEM((NOUT + K2, N), f32),
            pltpu.SemaphoreType.DMA((2 * T,)),
            pltpu.SemaphoreType.DMA((T,)),
        ],
    )(ar, ac, sr, sc, key, img)
    return out[None]


# speedup vs baseline: 2.0034x; 1.7321x over previous
import jax
import jax.numpy as jnp
from jax.experimental import pallas as pl
from jax.experimental.pallas import tpu as pltpu

N = 1024
C = 1024
KV = 128
KT = 128
KSEL = KV + KT
K2 = int((N - KSEL) * 0.25)
NOUT = KSEL + K2
SCALE = C ** -0.5
T = 4
TR = N // T


def _body(ar_ref, sr_ref, key_hbm, img_hbm, out_hbm,
          key_v, img_v, out_v, q_scr, in_sem, out_sem):
    f32 = jnp.float32

    key_cp = [pltpu.make_async_copy(key_hbm.at[t * TR:(t + 1) * TR, :],
                                    key_v.at[t * TR:(t + 1) * TR, :],
                                    in_sem.at[t]) for t in range(T)]
    img_cp = [pltpu.make_async_copy(img_hbm.at[t * TR:(t + 1) * TR, :],
                                    img_v.at[t * TR:(t + 1) * TR, :],
                                    in_sem.at[T + t]) for t in range(T)]
    for cp in key_cp:
        cp.start()
    for cp in img_cp:
        cp.start()

    ar = ar_ref[...]
    sr = sr_ref[...]

    ioj = jax.lax.broadcasted_iota(jnp.int32, (N, N), 0)
    ioi = jax.lax.broadcasted_iota(jnp.int32, (N, N), 1)
    ident = (ioj == ioi).astype(f32)

    def to_col(vr):
        return jax.lax.dot_general(ident, vr, (((1,), (1,)), ((), ())),
                                   preferred_element_type=f32)

    ac = to_col(ar)
    sc = to_col(sr)

    def rank_row(vc, vr):
        m = (vc > vr) | ((vc == vr) & (ioj < ioi))
        return jnp.sum(m.astype(f32), axis=0, keepdims=True)

    rv_r = rank_row(ac, ar)
    rt_r = rank_row(sc, sr)
    sel_r = ((rv_r < KV) | (rt_r < KT)).astype(f32)
    k1_r = sel_r * 1e6 + ar
    k1_c = to_col(k1_r)
    rs_r = rank_row(k1_c, k1_r)
    a_r = rs_r < KSEL
    a_c = to_col(a_r.astype(f32)) > 0.5

    m2 = (~a_c) & ((ac > ar) | ((ac == ar) & (ioj < ioi)))
    r2_r = jnp.sum(m2.astype(f32), axis=0, keepdims=True)
    b_r = (~a_r) & (r2_r < K2)
    cmask = (~a_r) & (~b_r)

    row_of = jnp.where(a_r, rs_r, jnp.where(b_r, KSEL + r2_r, 2.0 * N))
    io_out = jax.lax.broadcasted_iota(jnp.int32, (NOUT, N), 0)
    q_scr[0:NOUT, :] = (io_out == row_of.astype(jnp.int32)).astype(f32)

    neg = jnp.float32(-jnp.inf)
    t_log = jnp.where(cmask, 50.0 * sr, neg)
    te = jnp.exp(t_log - jnp.max(t_log, axis=1, keepdims=True))
    sm = te / jnp.sum(te, axis=1, keepdims=True)
    score = ar * sm

    kb = jnp.zeros((K2, C), f32)
    nk2_parts = []
    for t in range(T):
        key_cp[t].wait()
        key_t = key_v[t * TR:(t + 1) * TR, :]
        kb += jax.lax.dot_general(q_scr[KSEL:NOUT, t * TR:(t + 1) * TR],
                                  key_t, (((1,), (0,)), ((), ())),
                                  preferred_element_type=f32)
        nk2_parts.append(jnp.sum(key_t * key_t, axis=1, keepdims=True))
    nk2 = jnp.concatenate(nk2_parts, axis=0)

    cos_parts = []
    for t in range(T):
        key_t = key_v[t * TR:(t + 1) * TR, :]
        cos_parts.append(jax.lax.dot_general(kb, key_t,
                                             (((1,), (1,)), ((), ())),
                                             preferred_element_type=f32))
    cos = jnp.concatenate(cos_parts, axis=1)

    nk2_r = jax.lax.dot_general(nk2, ident, (((0,), (0,)), ((), ())),
                                preferred_element_type=f32)
    nb = jnp.maximum(jnp.sqrt(jnp.sum(kb * kb, axis=1, keepdims=True)),
                     1e-12)
    nk = jnp.maximum(jnp.sqrt(nk2_r), 1e-12)
    cosn = cos / nb / nk * SCALE
    logits = jnp.where(cmask, cosn, neg)
    e = jnp.exp(logits - jnp.max(logits, axis=1, keepdims=True))
    w = e / jnp.sum(e, axis=1, keepdims=True)
    q_scr[NOUT:NOUT + K2, :] = w * score

    for cp in img_cp:
        cp.wait()
    img = img_v[...]
    main = jax.lax.dot_general(q_scr[0:KSEL, :], img, (((1,), (0,)), ((), ())),
                               preferred_element_type=f32)
    out_v[0:KSEL, :] = main
    cp0 = pltpu.make_async_copy(out_v.at[0:KSEL, :], out_hbm.at[0:KSEL, :],
                                out_sem.at[0])
    cp0.start()
    rest = jax.lax.dot_general(q_scr[KSEL:NOUT + K2, :], img,
                               (((1,), (0,)), ((), ())),
                               preferred_element_type=f32)
    out_v[KSEL:NOUT, :] = rest[0:K2, :] + rest[K2:2 * K2, :]
    cp1 = pltpu.make_async_copy(out_v.at[KSEL:NOUT, :],
                                out_hbm.at[KSEL:NOUT, :], out_sem.at[1])
    cp1.start()
    cp0.wait()
    cp1.wait()


def kernel(image_features, key_features, cls_attn, similarity):
    img = image_features[0]
    key = key_features[0]
    f32 = jnp.float32
    out = pl.pallas_call(
        _body,
        in_specs=[
            pl.BlockSpec((1, N), lambda: (0, 0)),
            pl.BlockSpec((1, N), lambda: (0, 0)),
            pl.BlockSpec(memory_space=pl.ANY),
            pl.BlockSpec(memory_space=pl.ANY),
        ],
        out_specs=pl.BlockSpec(memory_space=pl.ANY),
        out_shape=jax.ShapeDtypeStruct((NOUT, C), f32),
        scratch_shapes=[
            pltpu.VMEM((N, C), f32),
            pltpu.VMEM((N, C), f32),
            pltpu.VMEM((NOUT, C), f32),
            pltpu.VMEM((NOUT + K2, N), f32),
            pltpu.SemaphoreType.DMA((2 * T,)),
            pltpu.SemaphoreType.DMA((2,)),
        ],
    )(cls_attn, similarity, key, img)
    return out[None]
